# Initial kernel scaffold; baseline (speedup 1.0000x reference)
import jax
import jax.numpy as jnp
from jax.experimental import pallas as pl


def kernel(x, edge_index, W1, b1, W2, b2):
    raise NotImplementedError("write your pallas kernel here")


# speedup vs baseline: 11.9433x; 11.9433x over previous
import functools

import jax
import jax.numpy as jnp
from jax import lax
from jax.experimental import pallas as pl
from jax.experimental.pallas import tpu as pltpu
from jax.experimental.pallas import tpu_sc as plsc

N = 10000
E = 320000
D = 128
NC = 2
NS = 16
NW = NC * NS
EPW = E // NW
K = 80
RPS = 624
TAIL = N - NS * RPS
DW = 16

_sc_mesh = plsc.VectorSubcoreMesh(core_axis_name="c", subcore_axis_name="s")


def _deg_body(dst_hbm, ones_hbm, out_hbm, dstv, onesv, deg_sh, sem):
    c = lax.axis_index("c")
    s = lax.axis_index("s")
    wid = s * NC + c
    r0 = s * RPS
    pltpu.sync_copy(ones_hbm.at[pl.ds(r0, RPS)], deg_sh.at[pl.ds(r0, RPS)])

    @pl.when(s == 0)
    def _():
        pltpu.sync_copy(ones_hbm.at[pl.ds(NS * RPS, TAIL)],
                        deg_sh.at[pl.ds(NS * RPS, TAIL)])

    pltpu.sync_copy(ones_hbm.at[pl.ds(0, K)], onesv)
    plsc.subcore_barrier()

    base0 = wid * EPW

    def body(i, carry):
        pltpu.sync_copy(dst_hbm.at[pl.ds(base0 + i * K, K)], dstv)
        pltpu.sync_copy(onesv, deg_sh.at[dstv], add=True)
        return carry

    lax.fori_loop(0, EPW // K, body, 0)
    plsc.subcore_barrier()
    pltpu.sync_copy(deg_sh.at[pl.ds(r0, RPS)],
                    out_hbm.at[pl.ds(c * N + r0, RPS)])

    @pl.when(s == 0)
    def _():
        pltpu.sync_copy(deg_sh.at[pl.ds(NS * RPS, TAIL)],
                        out_hbm.at[pl.ds(c * N + NS * RPS, TAIL)])


_deg_call = functools.partial(
    pl.kernel,
    out_type=jax.ShapeDtypeStruct((NC * N, D), jnp.float32),
    mesh=_sc_mesh,
    scratch_types=[
        pltpu.VMEM((K,), jnp.int32),
        pltpu.VMEM((K, D), jnp.float32),
        pltpu.VMEM_SHARED((N, D), jnp.float32),
        pltpu.SemaphoreType.DMA,
    ],
)(_deg_body)


def _agg_body(table_hbm, src_hbm, dst_hbm, out_hbm, srcv, dstv, rowsv, acc_sh,
              sem):
    c = lax.axis_index("c")
    s = lax.axis_index("s")
    wid = s * NC + c
    r0 = s * RPS
    pltpu.sync_copy(table_hbm.at[pl.ds(r0, RPS)], acc_sh.at[pl.ds(r0, RPS)])

    @pl.when(s == 0)
    def _():
        pltpu.sync_copy(table_hbm.at[pl.ds(NS * RPS, TAIL)],
                        acc_sh.at[pl.ds(NS * RPS, TAIL)])

    plsc.subcore_barrier()

    base0 = wid * EPW

    def body(i, carry):
        base = base0 + i * K
        pltpu.sync_copy(src_hbm.at[pl.ds(base, K)], srcv)
        pltpu.async_copy(table_hbm.at[srcv], rowsv, sem).wait()
        pltpu.sync_copy(dst_hbm.at[pl.ds(base, K)], dstv)
        pltpu.sync_copy(rowsv, acc_sh.at[dstv], add=True)
        return carry

    lax.fori_loop(0, EPW // K, body, 0)
    plsc.subcore_barrier()
    pltpu.sync_copy(acc_sh.at[pl.ds(r0, RPS)],
                    out_hbm.at[pl.ds(c * N + r0, RPS)])

    @pl.when(s == 0)
    def _():
        pltpu.sync_copy(acc_sh.at[pl.ds(NS * RPS, TAIL)],
                        out_hbm.at[pl.ds(c * N + NS * RPS, TAIL)])


_agg_call = functools.partial(
    pl.kernel,
    out_type=jax.ShapeDtypeStruct((NC * N, D), jnp.float32),
    mesh=_sc_mesh,
    scratch_types=[
        pltpu.VMEM((K,), jnp.int32),
        pltpu.VMEM((K,), jnp.int32),
        pltpu.VMEM((K, D), jnp.float32),
        pltpu.VMEM_SHARED((N, D), jnp.float32),
        pltpu.SemaphoreType.DMA,
    ],
)(_agg_body)


BR = 1000

def _dis(d0_ref, d1_ref):
    deg = d0_ref[:, 0:1] + d1_ref[:, 0:1] - 1.0
    return lax.rsqrt(deg)


def _tc1_body(x_ref, w_ref, d0_ref, d1_ref, o_ref):
    h = jnp.dot(x_ref[...], w_ref[...], preferred_element_type=jnp.float32)
    o_ref[...] = h * _dis(d0_ref, d1_ref)


def _tc2_body(p0_ref, p1_ref, t_ref, d0_ref, d1_ref, b_ref, w_ref, o_ref):
    dis = _dis(d0_ref, d1_ref)
    z = dis * (p0_ref[...] + p1_ref[...] - t_ref[...]) + b_ref[...]
    z = jnp.maximum(z, 0.0)
    o_ref[...] = jnp.dot(z, w_ref[...],
                         preferred_element_type=jnp.float32) * dis


def _tc3_body(q0_ref, q1_ref, t_ref, d0_ref, d1_ref, b_ref, o_ref):
    dis = _dis(d0_ref, d1_ref)
    o_ref[...] = dis * (q0_ref[...] + q1_ref[...] - t_ref[...]) + b_ref[...]


_row = pl.BlockSpec((BR, D), lambda i: (i, 0))
_deg_blk = pl.BlockSpec((BR, D), lambda i: (i, 0))
_wfull = pl.BlockSpec((D, D), lambda i: (0, 0))
_bfull = pl.BlockSpec((1, D), lambda i: (0, 0))
_grid = (N // BR,)
_out_rows = jax.ShapeDtypeStruct((N, D), jnp.float32)

_tc1 = pl.pallas_call(
    _tc1_body, grid=_grid,
    in_specs=[_row, _wfull, _deg_blk, _deg_blk],
    out_specs=_row, out_shape=_out_rows)

_tc2 = pl.pallas_call(
    _tc2_body, grid=_grid,
    in_specs=[_row, _row, _row, _deg_blk, _deg_blk, _bfull, _wfull],
    out_specs=_row, out_shape=_out_rows)

_tc3 = pl.pallas_call(
    _tc3_body, grid=_grid,
    in_specs=[_row, _row, _row, _deg_blk, _deg_blk, _bfull],
    out_specs=_row, out_shape=_out_rows)


@jax.jit
def kernel(x, edge_index, W1, b1, W2, b2):
    src = edge_index[0].astype(jnp.int32)
    dst = edge_index[1].astype(jnp.int32)
    ones = jnp.ones((N, D), jnp.float32)

    degp = _deg_call(dst, ones)
    d0, d1 = degp[:N], degp[N:]

    b1r = b1.reshape(1, D)
    b2r = b2.reshape(1, D)

    h1p = _tc1(x, W1, d0, d1)
    p = _agg_call(h1p, src, dst)
    h2p = _tc2(p[:N], p[N:], h1p, d0, d1, b1r, W2)
    q = _agg_call(h2p, src, dst)
    return _tc3(q[:N], q[N:], h2p, d0, d1, b2r)
